# baseline (speedup 1.0000x reference)
import functools
import math

import jax
import jax.numpy as jnp
from jax import lax
from jax.experimental import pallas as pl
from jax.experimental.pallas import tpu as pltpu
from jax.experimental.pallas import tpu_sc as plsc

E = 320000
D = 128
C = 50000
M = 200000

NSUB = 16
NCORE = 2
CHUNK = 12288
NCHUNK = 28
EPAD = CHUNK * NCHUNK
ROWS_PER_SUB = CHUNK // NSUB
MPAD = 204800
MSG_PER_SUB = MPAD // NSUB
B = 128
ACC_ROWS = CHUNK + 16


def _sc_gather_scatter(bond, cycle, i0, i1, src, dst, zeros_blk):
    mesh = plsc.VectorSubcoreMesh(core_axis_name="c", subcore_axis_name="s")

    @functools.partial(
        pl.kernel,
        mesh=mesh,
        out_type=jax.ShapeDtypeStruct((EPAD, D), jnp.float32),
        scratch_types=[
            pltpu.VMEM((B,), jnp.int32),
            pltpu.VMEM((B,), jnp.int32),
            pltpu.VMEM((B,), jnp.int32),
            pltpu.VMEM((B, D), jnp.float32),
            pltpu.VMEM_SHARED((ACC_ROWS, D), jnp.float32),
            pltpu.SemaphoreType.DMA,
        ],
    )
    def k(bond_h, cycle_h, i0_h, i1_h, src_h, dst_h, zeros_h, out_h,
          ia, ib, li, rows, acc, sem):
        core = lax.axis_index("c")
        s = lax.axis_index("s")

        def chunk_body(j, _):
            kk = j * NCORE + core
            lo = kk * CHUNK

            pltpu.sync_copy(zeros_h, acc.at[pl.ds(s * ROWS_PER_SUB, ROWS_PER_SUB)])
            plsc.subcore_barrier()

            def msg_body(b, _):
                base = s * MSG_PER_SUB + b * B
                pltpu.sync_copy(src_h.at[pl.ds(base, B)], ia)
                pltpu.sync_copy(dst_h.at[pl.ds(base, B)], ib)

                def lane(q, _):
                    d = ib[pl.ds(q * 16, 16)]
                    ok = (d >= lo) & (d < lo + CHUNK)
                    li[pl.ds(q * 16, 16)] = jnp.where(ok, d - lo, CHUNK)
                    return 0
                lax.fori_loop(0, B // 16, lane, 0)

                pltpu.async_copy(cycle_h.at[ia], rows, sem).wait()
                pltpu.sync_copy(rows, acc.at[li], add=True)
                return 0
            lax.fori_loop(0, MSG_PER_SUB // B, msg_body, 0)

            def bond_body(b, _):
                ebase = lo + s * ROWS_PER_SUB + b * B
                pltpu.sync_copy(i0_h.at[pl.ds(ebase, B)], ia)
                pltpu.sync_copy(i1_h.at[pl.ds(ebase, B)], ib)

                def lane2(q, _):
                    li[pl.ds(q * 16, 16)] = (
                        lax.broadcasted_iota(jnp.int32, (16,), 0)
                        + (s * ROWS_PER_SUB + b * B + q * 16)
                    )
                    return 0
                lax.fori_loop(0, B // 16, lane2, 0)

                pltpu.async_copy(bond_h.at[ia], rows, sem).wait()
                pltpu.sync_copy(rows, acc.at[li], add=True)
                pltpu.async_copy(bond_h.at[ib], rows, sem).wait()
                pltpu.sync_copy(rows, acc.at[li], add=True)
                return 0
            lax.fori_loop(0, ROWS_PER_SUB // B, bond_body, 0)

            plsc.subcore_barrier()
            pltpu.sync_copy(
                acc.at[pl.ds(s * ROWS_PER_SUB, ROWS_PER_SUB)],
                out_h.at[pl.ds(lo + s * ROWS_PER_SUB, ROWS_PER_SUB)],
            )
            plsc.subcore_barrier()
            return 0

        lax.fori_loop(0, NCHUNK // NCORE, chunk_body, 0)

    return k(bond, cycle, i0, i1, src, dst, zeros_blk)


BLK = 4000
NBLK = E // BLK


def _mm(a, w):
    return lax.dot_general(a, w, (((1,), (1,)), ((), ())),
                           preferred_element_type=jnp.float32)


def _pass1_body(x_ref, g_ref, we_ref, we2_ref, wl_ref, wl2_ref,
                t_ref, sums_ref, acc_ref):
    i = pl.program_id(0)
    h = _mm(_mm(x_ref[:], we_ref[:]), we2_ref[:]) + g_ref[:]
    y = _mm(_mm(h, wl_ref[:]), wl2_ref[:])
    t_ref[:] = y

    @pl.when(i == 0)
    def _():
        acc_ref[:] = jnp.zeros_like(acc_ref)

    acc_ref[0, :] += jnp.sum(y, axis=0)
    acc_ref[1, :] += jnp.sum(y * y, axis=0)

    @pl.when(i == NBLK - 1)
    def _():
        sums_ref[:] = acc_ref[:]


def _pass2_body(t_ref, sums_ref, gamma_ref, beta_ref, out_ref):
    mean = sums_ref[0:1, :] * (1.0 / E)
    var = sums_ref[1:2, :] * (1.0 / E) - mean * mean
    inv = lax.rsqrt(var + 1e-5)
    xn = (t_ref[:] - mean) * inv * gamma_ref[0:1, :] + beta_ref[0:1, :]
    out_ref[:] = xn * 0.5 * (1.0 + lax.erf(xn * (1.0 / math.sqrt(2.0))))


def kernel(x, bond, cycle, W_edge, W_edge2, W_lin, W_lin2,
           bn_gamma, bn_beta, edge_index, cycle_info):
    i0 = jnp.concatenate([edge_index[0], jnp.zeros((EPAD - E,), jnp.int32)])
    i1 = jnp.concatenate([edge_index[1], jnp.zeros((EPAD - E,), jnp.int32)])
    src = jnp.concatenate([cycle_info[2], jnp.zeros((MPAD - M,), jnp.int32)])
    dst = jnp.concatenate([cycle_info[3],
                           jnp.full((MPAD - M,), jnp.int32(2 ** 30))])
    zeros_blk = jnp.zeros((ROWS_PER_SUB, D), jnp.float32)

    g_full = _sc_gather_scatter(bond, cycle, i0, i1, src, dst, zeros_blk)
    g = g_full[:E]

    t, sums = pl.pallas_call(
        _pass1_body,
        grid=(NBLK,),
        in_specs=[
            pl.BlockSpec((BLK, D), lambda i: (i, 0)),
            pl.BlockSpec((BLK, D), lambda i: (i, 0)),
            pl.BlockSpec((D, D), lambda i: (0, 0)),
            pl.BlockSpec((D, D), lambda i: (0, 0)),
            pl.BlockSpec((D, D), lambda i: (0, 0)),
            pl.BlockSpec((D, D), lambda i: (0, 0)),
        ],
        out_specs=[
            pl.BlockSpec((BLK, D), lambda i: (i, 0)),
            pl.BlockSpec((8, D), lambda i: (0, 0)),
        ],
        out_shape=[
            jax.ShapeDtypeStruct((E, D), jnp.float32),
            jax.ShapeDtypeStruct((8, D), jnp.float32),
        ],
        scratch_shapes=[pltpu.VMEM((8, D), jnp.float32)],
    )(x, g, W_edge, W_edge2, W_lin, W_lin2)

    out = pl.pallas_call(
        _pass2_body,
        grid=(NBLK,),
        in_specs=[
            pl.BlockSpec((BLK, D), lambda i: (i, 0)),
            pl.BlockSpec((8, D), lambda i: (0, 0)),
            pl.BlockSpec((1, D), lambda i: (0, 0)),
            pl.BlockSpec((1, D), lambda i: (0, 0)),
        ],
        out_specs=pl.BlockSpec((BLK, D), lambda i: (i, 0)),
        out_shape=jax.ShapeDtypeStruct((E, D), jnp.float32),
    )(t, sums, bn_gamma.reshape(1, D), bn_beta.reshape(1, D))

    return out
